# baseline (speedup 1.0000x reference)
import functools

import jax
import jax.numpy as jnp
from jax import lax
from jax.experimental import pallas as pl
from jax.experimental.pallas import tpu as pltpu
from jax.experimental.pallas import tpu_sc as plsc

B = 16384
F = 26
NC = 2
NS = 16
NW = NC * NS
BPW = B // NW
IPW = BPW * F
L = 16
NCH = 4
CB = BPW // NCH
CI = CB * F


def _make_body(bpw):
    ipw = bpw * F

    def _body(x_hbm, table_hbm, bias_hbm, out_hbm, idx_v, vals_v, out_v,
              bias_v, sems):
        wid = lax.axis_index("s") * NC + lax.axis_index("c")
        base = wid * ipw
        pltpu.sync_copy(x_hbm.at[pl.ds(base, ipw)], idx_v)
        pltpu.sync_copy(bias_hbm, bias_v)
        bias_vec = bias_v[...]

        pltpu.async_copy(table_hbm.at[idx_v], vals_v, sems).wait()

        def block(j, _):
            off = j * L
            acc = bias_vec
            for f in range(F):
                acc = acc + vals_v[pl.ds(f * bpw + off, L)]
            out_v[pl.ds(off, L)] = 1.0 / (1.0 + jnp.exp(-acc))
            return 0

        lax.fori_loop(0, bpw // L, block, 0)

        pltpu.sync_copy(out_v, out_hbm.at[pl.ds(wid * bpw, bpw)])

    return _body


@functools.partial(jax.jit, static_argnames=())
def kernel(x, emb_table, bias):
    table_flat = jnp.pad(emb_table, ((0, 448), (0, 0))).reshape(-1)
    bias16 = jnp.broadcast_to(bias, (L,))
    mesh = plsc.VectorSubcoreMesh(core_axis_name="c", subcore_axis_name="s")

    nsplit = 2
    bh = B // nsplit
    bpw = bh // NW
    ipw = bpw * F
    call = pl.kernel(
        _make_body(bpw),
        mesh=mesh,
        out_type=jax.ShapeDtypeStruct((bh,), jnp.float32),
        scratch_types=[
            pltpu.VMEM((ipw,), jnp.int32),
            pltpu.VMEM((ipw,), jnp.float32),
            pltpu.VMEM((bpw,), jnp.float32),
            pltpu.VMEM((L,), jnp.float32),
            pltpu.SemaphoreType.DMA,
        ],
    )
    outs = []
    for s in range(nsplit):
        xs = lax.slice_in_dim(x, s * bh, (s + 1) * bh, axis=0)
        xs_flat = xs.reshape(NW, bpw, F).transpose(0, 2, 1).reshape(-1)
        outs.append(call(xs_flat, table_flat, bias16))
    return jnp.concatenate(outs).reshape(B, 1)


# speedup vs baseline: 1.1666x; 1.1666x over previous
import functools

import jax
import jax.numpy as jnp
from jax import lax
from jax.experimental import pallas as pl
from jax.experimental.pallas import tpu as pltpu
from jax.experimental.pallas import tpu_sc as plsc

B = 16384
F = 26
NC = 2
NS = 16
NW = NC * NS
BPW = B // NW
IPW = BPW * F
L = 16


def _body(x_hbm, table_hbm, bias_hbm, out_hbm, idx_v, vals_v, out_v, bias_v,
          sem):
    wid = lax.axis_index("s") * NC + lax.axis_index("c")
    base = wid * IPW
    pltpu.sync_copy(x_hbm.at[pl.ds(base, IPW)], idx_v)
    pltpu.sync_copy(bias_hbm, bias_v)
    bias_vec = bias_v[...]

    pltpu.async_copy(table_hbm.at[idx_v], vals_v, sem).wait()

    @functools.partial(plsc.parallel_loop, 0, BPW // L, unroll=2)
    def _block(j):
        off = j * L
        acc = bias_vec
        for f in range(F):
            acc = acc + vals_v[pl.ds(f * BPW + off, L)]
        out_v[pl.ds(off, L)] = 1.0 / (1.0 + jnp.exp(-acc))

    pltpu.sync_copy(out_v, out_hbm.at[pl.ds(wid * BPW, BPW)])


@functools.partial(jax.jit, static_argnames=())
def kernel(x, emb_table, bias):
    x_flat = x.reshape(NW, BPW, F).transpose(0, 2, 1).reshape(-1)
    table_flat = jnp.pad(emb_table, ((0, 448), (0, 0))).reshape(-1)
    bias16 = jnp.broadcast_to(bias, (L,))
    mesh = plsc.VectorSubcoreMesh(core_axis_name="c", subcore_axis_name="s")
    out = pl.kernel(
        _body,
        mesh=mesh,
        out_type=jax.ShapeDtypeStruct((B,), jnp.float32),
        scratch_types=[
            pltpu.VMEM((IPW,), jnp.int32),
            pltpu.VMEM((IPW,), jnp.float32),
            pltpu.VMEM((BPW,), jnp.float32),
            pltpu.VMEM((L,), jnp.float32),
            pltpu.SemaphoreType.DMA,
        ],
    )(x_flat, table_flat, bias16)
    return out.reshape(B, 1)
